# baseline (speedup 1.0000x reference)
import functools
import jax
import jax.numpy as jnp
from jax import lax
from jax.experimental import pallas as pl
from jax.experimental.pallas import tpu as pltpu
from jax.experimental.pallas import tpu_sc as plsc


def _sc_w_kernel(src_hbm, dst_hbm, ew_hbm, w_hbm, src_v, dst_v, ew_v, w_v):
    f32 = jnp.float32
    wid = lax.axis_index("s") * 2 + lax.axis_index("c")
    pltpu.sync_copy(ew_hbm, ew_v)
    for c in range(4):
        g0 = wid * 64 + c * 16
        ebase = g0 * 256
        nbase = g0 * 64

        pltpu.sync_copy(src_hbm.at[pl.ds(ebase, 4096)], src_v)
        pltpu.sync_copy(dst_hbm.at[pl.ds(ebase, 4096)], dst_v)

        def zbody(j, carry):
            w_v[pl.ds(pl.multiple_of(j * 16, 16), 16)] = jnp.zeros((16,), f32)
            return carry

        lax.fori_loop(0, 4096, zbody, 0)

        def sbody(j, carry):
            off = pl.multiple_of(j * 16, 16)
            s = src_v[pl.ds(off, 16)]
            d = dst_v[pl.ds(off, 16)]
            ewv = ew_v[pl.ds(pl.multiple_of((j % 16) * 16, 16), 16)]
            idx = (d - nbase) * 64 + (s & 63)
            plsc.addupdate_scatter(w_v, [idx], ewv)
            return carry

        lax.fori_loop(0, 256, sbody, 0)
        pltpu.sync_copy(w_v, w_hbm.at[pl.ds(nbase * 64, 65536)])


def _celu(v):
    return jnp.where(v > 0, v, jnp.exp(jnp.minimum(v, 0.0)) - 1.0)


def _stats_kernel(x_ref, gamma_ref, beta_ref, scale_ref, shift_ref, acc_ref, *, nsteps, n_rows):
    i = pl.program_id(0)

    @pl.when(i == 0)
    def _init():
        acc_ref[...] = jnp.zeros_like(acc_ref)

    xb = x_ref[...]
    acc_ref[0:1, :] += jnp.sum(xb, axis=0, keepdims=True)
    acc_ref[1:2, :] += jnp.sum(xb * xb, axis=0, keepdims=True)

    @pl.when(i == nsteps - 1)
    def _fin():
        inv_n = 1.0 / n_rows
        mean = acc_ref[0:1, :] * inv_n
        var = acc_ref[1:2, :] * inv_n - mean * mean
        rstd = jax.lax.rsqrt(var + 1e-5)
        sc = gamma_ref[...] * rstd
        scale_ref[...] = sc
        shift_ref[...] = beta_ref[...] - mean * sc


def _gcn_kernel(x_ref, src_ref, dst_ref, ew_ref, wsc_ref, scale_ref, shift_ref,
                w1_ref, b1_ref, w2_ref, b2_ref, out_ref, *, bt, n_per, e_per):
    f32 = jnp.float32
    xb = x_ref[...] * scale_ref[...] + shift_ref[...]
    h1 = jnp.dot(xb, w1_ref[...], preferred_element_type=f32)
    ew = ew_ref[...]
    iota_n = jax.lax.broadcasted_iota(jnp.int32, (n_per, e_per), 0)
    pos = (jax.lax.broadcasted_iota(jnp.int32, (n_per, n_per), 0) * n_per
           + jax.lax.broadcasted_iota(jnp.int32, (n_per, n_per), 1)).astype(f32)
    li = jax.lax.broadcasted_iota(jnp.int32, (1, n_per * n_per), 1).astype(f32)
    diffs = []
    wmats = []
    dinvs = []
    for b in range(bt):
        src = src_ref[0, :, b * e_per:(b + 1) * e_per] & (n_per - 1)
        dst = dst_ref[0, :, b * e_per:(b + 1) * e_per] & (n_per - 1)
        dt = jnp.where(iota_n == dst, 1.0, 0.0)
        stw = jnp.where(iota_n == src, ew, 0.0)
        wmat = jax.lax.dot_general(dt, stw, (((1,), (1,)), ((), ())),
                                   preferred_element_type=f32)
        deg = jnp.sum(wmat, axis=1, keepdims=True) + 1.0
        dinvs.append(jnp.where(deg > 0, jax.lax.rsqrt(deg), 0.0))
        wmats.append(wmat)
        wsl = wsc_ref[0, :, b * n_per * n_per:(b + 1) * n_per * n_per]
        diffs.append(jnp.abs(jnp.sum(wsl) - jnp.sum(wmat))
                     + jnp.abs(jnp.sum(wsl * li) - jnp.sum(wmat * pos)))
    amp = 1e3 * sum(diffs)
    dinv = jnp.concatenate(dinvs, axis=0)
    t1 = dinv * h1
    u1 = jnp.concatenate(
        [jnp.dot(wmats[b], t1[b * n_per:(b + 1) * n_per, :], preferred_element_type=f32)
         for b in range(bt)], axis=0)
    c1 = _celu(dinv * (u1 + t1) + b1_ref[...])
    h2 = jnp.dot(c1, w2_ref[...], preferred_element_type=f32)
    t2 = dinv * h2
    u2 = jnp.concatenate(
        [jnp.dot(wmats[b], t2[b * n_per:(b + 1) * n_per, :], preferred_element_type=f32)
         for b in range(bt)], axis=0)
    out_ref[...] = _celu(dinv * (u2 + t2) + b2_ref[...]) + amp


def _fc_kernel(g_ref, w1_ref, b1_ref, w2_ref, b2_ref, out_ref, *, n_per):
    f32 = jnp.float32
    gv = jnp.concatenate([g_ref[:, nn, :] for nn in range(n_per)], axis=1)
    h = _celu(jnp.dot(gv, w1_ref[...], preferred_element_type=f32) + b1_ref[...])
    out_ref[...] = jnp.dot(h, w2_ref[...], preferred_element_type=f32) + b2_ref[...]


@jax.jit
def kernel(x, edge_index, edge_weight, bn_gamma, bn_beta, W1, b1, W2, b2, fc1_W, fc1_b, fc2_W, fc2_b):
    f32 = jnp.float32
    n, f = x.shape
    g1 = W1.shape[1]
    g2 = W2.shape[1]
    n_per = fc1_W.shape[0] // g2
    b_graphs = n // n_per
    e_per = edge_weight.shape[0]

    bt = 16 if b_graphs % 16 == 0 else 1
    nsteps = b_graphs // bt
    rows = bt * n_per

    gamma2 = bn_gamma.reshape(1, f)
    beta2 = bn_beta.reshape(1, f)
    scale, shift = pl.pallas_call(
        functools.partial(_stats_kernel, nsteps=nsteps, n_rows=float(n)),
        grid=(nsteps,),
        in_specs=[
            pl.BlockSpec((rows, f), lambda i: (i, 0)),
            pl.BlockSpec((1, f), lambda i: (0, 0)),
            pl.BlockSpec((1, f), lambda i: (0, 0)),
        ],
        out_specs=[
            pl.BlockSpec((1, f), lambda i: (0, 0)),
            pl.BlockSpec((1, f), lambda i: (0, 0)),
        ],
        out_shape=[
            jax.ShapeDtypeStruct((1, f), f32),
            jax.ShapeDtypeStruct((1, f), f32),
        ],
        scratch_shapes=[pltpu.VMEM((2, f), f32)],
    )(x, gamma2, beta2)

    src3 = edge_index[0].reshape(nsteps, 1, bt * e_per)
    dst3 = edge_index[1].reshape(nsteps, 1, bt * e_per)
    ew2 = edge_weight.reshape(1, e_per)

    wflat = pl.kernel(
        _sc_w_kernel,
        out_type=jax.ShapeDtypeStruct((b_graphs * n_per * n_per,), f32),
        mesh=plsc.VectorSubcoreMesh(core_axis_name="c", subcore_axis_name="s"),
        compiler_params=pltpu.CompilerParams(needs_layout_passes=False),
        scratch_types=[
            pltpu.VMEM((4096,), jnp.int32),
            pltpu.VMEM((4096,), jnp.int32),
            pltpu.VMEM((e_per,), f32),
            pltpu.VMEM((65536,), f32),
        ],
    )(edge_index[0], edge_index[1], edge_weight)
    w3 = wflat.reshape(nsteps, 1, bt * n_per * n_per)

    c2 = pl.pallas_call(
        functools.partial(_gcn_kernel, bt=bt, n_per=n_per, e_per=e_per),
        grid=(nsteps,),
        in_specs=[
            pl.BlockSpec((rows, f), lambda i: (i, 0)),
            pl.BlockSpec((1, 1, bt * e_per), lambda i: (i, 0, 0)),
            pl.BlockSpec((1, 1, bt * e_per), lambda i: (i, 0, 0)),
            pl.BlockSpec((1, e_per), lambda i: (0, 0)),
            pl.BlockSpec((1, 1, bt * n_per * n_per), lambda i: (i, 0, 0)),
            pl.BlockSpec((1, f), lambda i: (0, 0)),
            pl.BlockSpec((1, f), lambda i: (0, 0)),
            pl.BlockSpec((f, g1), lambda i: (0, 0)),
            pl.BlockSpec((1, g1), lambda i: (0, 0)),
            pl.BlockSpec((g1, g2), lambda i: (0, 0)),
            pl.BlockSpec((1, g2), lambda i: (0, 0)),
        ],
        out_specs=pl.BlockSpec((rows, g2), lambda i: (i, 0)),
        out_shape=jax.ShapeDtypeStruct((n, g2), f32),
    )(x, src3, dst3, ew2, w3, scale, shift, W1, b1.reshape(1, g1), W2, b2.reshape(1, g2))

    g = c2.reshape(b_graphs, n_per, g2)

    fc1_n = fc1_W.shape[1]
    out_n = fc2_W.shape[1]
    fc_bt = 256 if b_graphs % 256 == 0 else b_graphs
    logits = pl.pallas_call(
        functools.partial(_fc_kernel, n_per=n_per),
        grid=(b_graphs // fc_bt,),
        in_specs=[
            pl.BlockSpec((fc_bt, n_per, g2), lambda i: (i, 0, 0)),
            pl.BlockSpec((n_per * g2, fc1_n), lambda i: (0, 0)),
            pl.BlockSpec((1, fc1_n), lambda i: (0, 0)),
            pl.BlockSpec((fc1_n, out_n), lambda i: (0, 0)),
            pl.BlockSpec((1, out_n), lambda i: (0, 0)),
        ],
        out_specs=pl.BlockSpec((fc_bt, out_n), lambda i: (i, 0)),
        out_shape=jax.ShapeDtypeStruct((b_graphs, out_n), f32),
    )(g, fc1_W, fc1_b.reshape(1, fc1_n), fc2_W, fc2_b.reshape(1, out_n))
    return logits


# speedup vs baseline: 1.1997x; 1.1997x over previous
import functools
import jax
import jax.numpy as jnp
from jax.experimental import pallas as pl
from jax.experimental.pallas import tpu as pltpu


def _celu(v):
    return jnp.where(v > 0, v, jnp.exp(jnp.minimum(v, 0.0)) - 1.0)


def _stats_kernel(x_ref, gamma_ref, beta_ref, scale_ref, shift_ref, acc_ref, *, nsteps, n_rows):
    i = pl.program_id(0)

    @pl.when(i == 0)
    def _init():
        acc_ref[...] = jnp.zeros_like(acc_ref)

    xb = x_ref[...]
    acc_ref[0:1, :] += jnp.sum(xb, axis=0, keepdims=True)
    acc_ref[1:2, :] += jnp.sum(xb * xb, axis=0, keepdims=True)

    @pl.when(i == nsteps - 1)
    def _fin():
        inv_n = 1.0 / n_rows
        mean = acc_ref[0:1, :] * inv_n
        var = acc_ref[1:2, :] * inv_n - mean * mean
        rstd = jax.lax.rsqrt(var + 1e-5)
        sc = gamma_ref[...] * rstd
        scale_ref[...] = sc
        shift_ref[...] = beta_ref[...] - mean * sc


def _gcn_kernel(x_ref, src_ref, dst_ref, ew_ref, scale_ref, shift_ref,
                w1_ref, b1_ref, w2_ref, b2_ref, out_ref, *, bt, n_per, e_per):
    f32 = jnp.float32
    xb = x_ref[...] * scale_ref[...] + shift_ref[...]
    h1 = jnp.dot(xb, w1_ref[...], preferred_element_type=f32)
    ew = ew_ref[...]
    iota_n = jax.lax.broadcasted_iota(jnp.int32, (n_per, e_per), 0)
    wmats = []
    dinvs = []
    for b in range(bt):
        src = src_ref[0, :, b * e_per:(b + 1) * e_per] & (n_per - 1)
        dst = dst_ref[0, :, b * e_per:(b + 1) * e_per] & (n_per - 1)
        dt = jnp.where(iota_n == dst, 1.0, 0.0)
        stw = jnp.where(iota_n == src, ew, 0.0)
        wmat = jax.lax.dot_general(dt, stw, (((1,), (1,)), ((), ())),
                                   preferred_element_type=f32)
        deg = jnp.sum(wmat, axis=1, keepdims=True) + 1.0
        dinvs.append(jnp.where(deg > 0, jax.lax.rsqrt(deg), 0.0))
        wmats.append(wmat)
    dinv = jnp.concatenate(dinvs, axis=0)
    t1 = dinv * h1
    u1 = jnp.concatenate(
        [jnp.dot(wmats[b], t1[b * n_per:(b + 1) * n_per, :], preferred_element_type=f32)
         for b in range(bt)], axis=0)
    c1 = _celu(dinv * (u1 + t1) + b1_ref[...])
    h2 = jnp.dot(c1, w2_ref[...], preferred_element_type=f32)
    t2 = dinv * h2
    u2 = jnp.concatenate(
        [jnp.dot(wmats[b], t2[b * n_per:(b + 1) * n_per, :], preferred_element_type=f32)
         for b in range(bt)], axis=0)
    out_ref[...] = _celu(dinv * (u2 + t2) + b2_ref[...])


def _fc_kernel(g_ref, w1_ref, b1_ref, w2_ref, b2_ref, out_ref, *, n_per):
    f32 = jnp.float32
    gv = jnp.concatenate([g_ref[:, nn, :] for nn in range(n_per)], axis=1)
    h = _celu(jnp.dot(gv, w1_ref[...], preferred_element_type=f32) + b1_ref[...])
    out_ref[...] = jnp.dot(h, w2_ref[...], preferred_element_type=f32) + b2_ref[...]


@jax.jit
def kernel(x, edge_index, edge_weight, bn_gamma, bn_beta, W1, b1, W2, b2, fc1_W, fc1_b, fc2_W, fc2_b):
    f32 = jnp.float32
    n, f = x.shape
    g1 = W1.shape[1]
    g2 = W2.shape[1]
    n_per = fc1_W.shape[0] // g2
    b_graphs = n // n_per
    e_per = edge_weight.shape[0]

    bt = 16 if b_graphs % 16 == 0 else 1
    nsteps = b_graphs // bt
    rows = bt * n_per

    gamma2 = bn_gamma.reshape(1, f)
    beta2 = bn_beta.reshape(1, f)
    scale, shift = pl.pallas_call(
        functools.partial(_stats_kernel, nsteps=nsteps, n_rows=float(n)),
        grid=(nsteps,),
        in_specs=[
            pl.BlockSpec((rows, f), lambda i: (i, 0)),
            pl.BlockSpec((1, f), lambda i: (0, 0)),
            pl.BlockSpec((1, f), lambda i: (0, 0)),
        ],
        out_specs=[
            pl.BlockSpec((1, f), lambda i: (0, 0)),
            pl.BlockSpec((1, f), lambda i: (0, 0)),
        ],
        out_shape=[
            jax.ShapeDtypeStruct((1, f), f32),
            jax.ShapeDtypeStruct((1, f), f32),
        ],
        scratch_shapes=[pltpu.VMEM((2, f), f32)],
    )(x, gamma2, beta2)

    src3 = edge_index[0].reshape(nsteps, 1, bt * e_per)
    dst3 = edge_index[1].reshape(nsteps, 1, bt * e_per)
    ew2 = edge_weight.reshape(1, e_per)

    c2 = pl.pallas_call(
        functools.partial(_gcn_kernel, bt=bt, n_per=n_per, e_per=e_per),
        grid=(nsteps,),
        in_specs=[
            pl.BlockSpec((rows, f), lambda i: (i, 0)),
            pl.BlockSpec((1, 1, bt * e_per), lambda i: (i, 0, 0)),
            pl.BlockSpec((1, 1, bt * e_per), lambda i: (i, 0, 0)),
            pl.BlockSpec((1, e_per), lambda i: (0, 0)),
            pl.BlockSpec((1, f), lambda i: (0, 0)),
            pl.BlockSpec((1, f), lambda i: (0, 0)),
            pl.BlockSpec((f, g1), lambda i: (0, 0)),
            pl.BlockSpec((1, g1), lambda i: (0, 0)),
            pl.BlockSpec((g1, g2), lambda i: (0, 0)),
            pl.BlockSpec((1, g2), lambda i: (0, 0)),
        ],
        out_specs=pl.BlockSpec((rows, g2), lambda i: (i, 0)),
        out_shape=jax.ShapeDtypeStruct((n, g2), f32),
    )(x, src3, dst3, ew2, scale, shift, W1, b1.reshape(1, g1), W2, b2.reshape(1, g2))

    g = c2.reshape(b_graphs, n_per, g2)

    fc1_n = fc1_W.shape[1]
    out_n = fc2_W.shape[1]
    fc_bt = 256 if b_graphs % 256 == 0 else b_graphs
    logits = pl.pallas_call(
        functools.partial(_fc_kernel, n_per=n_per),
        grid=(b_graphs // fc_bt,),
        in_specs=[
            pl.BlockSpec((fc_bt, n_per, g2), lambda i: (i, 0, 0)),
            pl.BlockSpec((n_per * g2, fc1_n), lambda i: (0, 0)),
            pl.BlockSpec((1, fc1_n), lambda i: (0, 0)),
            pl.BlockSpec((fc1_n, out_n), lambda i: (0, 0)),
            pl.BlockSpec((1, out_n), lambda i: (0, 0)),
        ],
        out_specs=pl.BlockSpec((fc_bt, out_n), lambda i: (i, 0)),
        out_shape=jax.ShapeDtypeStruct((b_graphs, out_n), f32),
    )(g, fc1_W, fc1_b.reshape(1, fc1_n), fc2_W, fc2_b.reshape(1, out_n))
    return logits


# speedup vs baseline: 1.5763x; 1.3139x over previous
import functools
import jax
import jax.numpy as jnp
from jax.experimental import pallas as pl
from jax.experimental.pallas import tpu as pltpu


def _celu(v):
    return jnp.where(v > 0, v, jnp.exp(jnp.minimum(v, 0.0)) - 1.0)


def _stats_kernel(x_ref, gamma_ref, beta_ref, scale_ref, shift_ref, acc_ref, *, nsteps, n_rows):
    i = pl.program_id(0)

    @pl.when(i == 0)
    def _init():
        acc_ref[...] = jnp.zeros_like(acc_ref)

    xb = x_ref[...]
    acc_ref[0:1, :] += jnp.sum(xb, axis=0, keepdims=True)
    acc_ref[1:2, :] += jnp.sum(xb * xb, axis=0, keepdims=True)

    @pl.when(i == nsteps - 1)
    def _fin():
        inv_n = 1.0 / n_rows
        mean = acc_ref[0:1, :] * inv_n
        var = acc_ref[1:2, :] * inv_n - mean * mean
        rstd = jax.lax.rsqrt(var + 1e-5)
        sc = gamma_ref[...] * rstd
        scale_ref[...] = sc
        shift_ref[...] = beta_ref[...] - mean * sc


def _gcn_kernel(x_ref, src_ref, dst_ref, ew_ref, scale_ref, shift_ref,
                w1_ref, b1_ref, w2_ref, b2_ref, out_ref, *, bt, n_per, e_per):
    f32 = jnp.float32
    xb = x_ref[...] * scale_ref[...] + shift_ref[...]
    h1 = jnp.dot(xb, w1_ref[...], preferred_element_type=f32)
    ew = ew_ref[...]
    iota_n = jax.lax.broadcasted_iota(jnp.int32, (n_per, e_per), 0)
    wmats = []
    dinvs = []
    for b in range(bt):
        src = src_ref[0, :, b * e_per:(b + 1) * e_per] & (n_per - 1)
        dst = dst_ref[0, :, b * e_per:(b + 1) * e_per] & (n_per - 1)
        dt = jnp.where(iota_n == dst, 1.0, 0.0)
        stw = jnp.where(iota_n == src, ew, 0.0)
        wmat = jax.lax.dot_general(dt, stw, (((1,), (1,)), ((), ())),
                                   preferred_element_type=f32)
        deg = jnp.sum(wmat, axis=1, keepdims=True) + 1.0
        dinvs.append(jnp.where(deg > 0, jax.lax.rsqrt(deg), 0.0))
        wmats.append(wmat)
    dinv = jnp.concatenate(dinvs, axis=0)
    t1 = dinv * h1
    u1 = jnp.concatenate(
        [jnp.dot(wmats[b], t1[b * n_per:(b + 1) * n_per, :], preferred_element_type=f32)
         for b in range(bt)], axis=0)
    c1 = _celu(dinv * (u1 + t1) + b1_ref[...])
    h2 = jnp.dot(c1, w2_ref[...], preferred_element_type=f32)
    t2 = dinv * h2
    u2 = jnp.concatenate(
        [jnp.dot(wmats[b], t2[b * n_per:(b + 1) * n_per, :], preferred_element_type=f32)
         for b in range(bt)], axis=0)
    out_ref[...] = _celu(dinv * (u2 + t2) + b2_ref[...])


def _fc_kernel(g_ref, w1_ref, b1_ref, w2_ref, b2_ref, out_ref, *, n_per):
    f32 = jnp.float32
    gv = jnp.concatenate([g_ref[:, nn, :] for nn in range(n_per)], axis=1)
    h = _celu(jnp.dot(gv, w1_ref[...], preferred_element_type=f32) + b1_ref[...])
    out_ref[...] = jnp.dot(h, w2_ref[...], preferred_element_type=f32) + b2_ref[...]


@jax.jit
def kernel(x, edge_index, edge_weight, bn_gamma, bn_beta, W1, b1, W2, b2, fc1_W, fc1_b, fc2_W, fc2_b):
    f32 = jnp.float32
    n, f = x.shape
    g1 = W1.shape[1]
    g2 = W2.shape[1]
    n_per = fc1_W.shape[0] // g2
    b_graphs = n // n_per
    e_per = edge_weight.shape[0]

    bt = 32 if b_graphs % 32 == 0 else 1
    nsteps = b_graphs // bt
    rows = bt * n_per

    gamma2 = bn_gamma.reshape(1, f)
    beta2 = bn_beta.reshape(1, f)
    scale, shift = pl.pallas_call(
        functools.partial(_stats_kernel, nsteps=nsteps, n_rows=float(n)),
        grid=(nsteps,),
        in_specs=[
            pl.BlockSpec((rows, f), lambda i: (i, 0)),
            pl.BlockSpec((1, f), lambda i: (0, 0)),
            pl.BlockSpec((1, f), lambda i: (0, 0)),
        ],
        out_specs=[
            pl.BlockSpec((1, f), lambda i: (0, 0)),
            pl.BlockSpec((1, f), lambda i: (0, 0)),
        ],
        out_shape=[
            jax.ShapeDtypeStruct((1, f), f32),
            jax.ShapeDtypeStruct((1, f), f32),
        ],
        scratch_shapes=[pltpu.VMEM((2, f), f32)],
    )(x, gamma2, beta2)

    src3 = edge_index[0].reshape(nsteps, 1, bt * e_per)
    dst3 = edge_index[1].reshape(nsteps, 1, bt * e_per)
    ew2 = edge_weight.reshape(1, e_per)

    c2 = pl.pallas_call(
        functools.partial(_gcn_kernel, bt=bt, n_per=n_per, e_per=e_per),
        grid=(nsteps,),
        in_specs=[
            pl.BlockSpec((rows, f), lambda i: (i, 0)),
            pl.BlockSpec((1, 1, bt * e_per), lambda i: (i, 0, 0)),
            pl.BlockSpec((1, 1, bt * e_per), lambda i: (i, 0, 0)),
            pl.BlockSpec((1, e_per), lambda i: (0, 0)),
            pl.BlockSpec((1, f), lambda i: (0, 0)),
            pl.BlockSpec((1, f), lambda i: (0, 0)),
            pl.BlockSpec((f, g1), lambda i: (0, 0)),
            pl.BlockSpec((1, g1), lambda i: (0, 0)),
            pl.BlockSpec((g1, g2), lambda i: (0, 0)),
            pl.BlockSpec((1, g2), lambda i: (0, 0)),
        ],
        out_specs=pl.BlockSpec((rows, g2), lambda i: (i, 0)),
        out_shape=jax.ShapeDtypeStruct((n, g2), f32),
    )(x, src3, dst3, ew2, scale, shift, W1, b1.reshape(1, g1), W2, b2.reshape(1, g2))

    g = c2.reshape(b_graphs, n_per, g2)

    fc1_n = fc1_W.shape[1]
    out_n = fc2_W.shape[1]
    fc_bt = 256 if b_graphs % 256 == 0 else b_graphs
    logits = pl.pallas_call(
        functools.partial(_fc_kernel, n_per=n_per),
        grid=(b_graphs // fc_bt,),
        in_specs=[
            pl.BlockSpec((fc_bt, n_per, g2), lambda i: (i, 0, 0)),
            pl.BlockSpec((n_per * g2, fc1_n), lambda i: (0, 0)),
            pl.BlockSpec((1, fc1_n), lambda i: (0, 0)),
            pl.BlockSpec((fc1_n, out_n), lambda i: (0, 0)),
            pl.BlockSpec((1, out_n), lambda i: (0, 0)),
        ],
        out_specs=pl.BlockSpec((fc_bt, out_n), lambda i: (i, 0)),
        out_shape=jax.ShapeDtypeStruct((b_graphs, out_n), f32),
    )(g, fc1_W, fc1_b.reshape(1, fc1_n), fc2_W, fc2_b.reshape(1, out_n))
    return logits


# speedup vs baseline: 1.7584x; 1.1155x over previous
import functools
import jax
import jax.numpy as jnp
from jax.experimental import pallas as pl
from jax.experimental.pallas import tpu as pltpu


def _celu(v):
    return jnp.where(v > 0, v, jnp.exp(jnp.minimum(v, 0.0)) - 1.0)


def _stats_kernel(x_ref, gamma_ref, beta_ref, scale_ref, shift_ref, acc_ref, *, nsteps, n_rows):
    i = pl.program_id(0)

    @pl.when(i == 0)
    def _init():
        acc_ref[...] = jnp.zeros_like(acc_ref)

    xb = x_ref[...]
    acc_ref[0:1, :] += jnp.sum(xb, axis=0, keepdims=True)
    acc_ref[1:2, :] += jnp.sum(xb * xb, axis=0, keepdims=True)

    @pl.when(i == nsteps - 1)
    def _fin():
        inv_n = 1.0 / n_rows
        mean = acc_ref[0:1, :] * inv_n
        var = acc_ref[1:2, :] * inv_n - mean * mean
        rstd = jax.lax.rsqrt(var + 1e-5)
        sc = gamma_ref[...] * rstd
        scale_ref[...] = sc
        shift_ref[...] = beta_ref[...] - mean * sc


def _gcn_kernel(x_ref, src_ref, dst_ref, ew_ref, scale_ref, shift_ref,
                w1_ref, b1_ref, w2_ref, b2_ref, out_ref, *, bt, n_per, e_per):
    f32 = jnp.float32
    xb = x_ref[...] * scale_ref[...] + shift_ref[...]
    h1 = jnp.dot(xb, w1_ref[...], preferred_element_type=f32)
    ew = ew_ref[...]
    iota_n = jax.lax.broadcasted_iota(jnp.int32, (n_per, e_per), 0)
    wmats = []
    dinvs = []
    for b in range(bt):
        src = src_ref[0, :, b * e_per:(b + 1) * e_per] & (n_per - 1)
        dst = dst_ref[0, :, b * e_per:(b + 1) * e_per] & (n_per - 1)
        dt = jnp.where(iota_n == dst, 1.0, 0.0)
        stw = jnp.where(iota_n == src, ew, 0.0)
        wmat = jax.lax.dot_general(dt, stw, (((1,), (1,)), ((), ())),
                                   preferred_element_type=f32)
        deg = jnp.sum(wmat, axis=1, keepdims=True) + 1.0
        dinvs.append(jnp.where(deg > 0, jax.lax.rsqrt(deg), 0.0))
        wmats.append(wmat)
    dinv = jnp.concatenate(dinvs, axis=0)
    t1 = dinv * h1
    u1 = jnp.concatenate(
        [jnp.dot(wmats[b], t1[b * n_per:(b + 1) * n_per, :], preferred_element_type=f32)
         for b in range(bt)], axis=0)
    c1 = _celu(dinv * (u1 + t1) + b1_ref[...])
    h2 = jnp.dot(c1, w2_ref[...], preferred_element_type=f32)
    t2 = dinv * h2
    u2 = jnp.concatenate(
        [jnp.dot(wmats[b], t2[b * n_per:(b + 1) * n_per, :], preferred_element_type=f32)
         for b in range(bt)], axis=0)
    out_ref[...] = _celu(dinv * (u2 + t2) + b2_ref[...])


def _fc_kernel(g_ref, w1_ref, b1_ref, w2_ref, b2_ref, out_ref, *, n_per):
    f32 = jnp.float32
    gv = jnp.concatenate([g_ref[:, nn, :] for nn in range(n_per)], axis=1)
    h = _celu(jnp.dot(gv, w1_ref[...], preferred_element_type=f32) + b1_ref[...])
    out_ref[...] = jnp.dot(h, w2_ref[...], preferred_element_type=f32) + b2_ref[...]


@jax.jit
def kernel(x, edge_index, edge_weight, bn_gamma, bn_beta, W1, b1, W2, b2, fc1_W, fc1_b, fc2_W, fc2_b):
    f32 = jnp.float32
    n, f = x.shape
    g1 = W1.shape[1]
    g2 = W2.shape[1]
    n_per = fc1_W.shape[0] // g2
    b_graphs = n // n_per
    e_per = edge_weight.shape[0]

    bt = 32 if b_graphs % 32 == 0 else 1
    nsteps = b_graphs // bt
    rows = bt * n_per

    gamma2 = bn_gamma.reshape(1, f)
    beta2 = bn_beta.reshape(1, f)
    srows = 8192 if n % 8192 == 0 else rows
    s_steps = n // srows
    scale, shift = pl.pallas_call(
        functools.partial(_stats_kernel, nsteps=s_steps, n_rows=float(n)),
        grid=(s_steps,),
        in_specs=[
            pl.BlockSpec((srows, f), lambda i: (i, 0)),
            pl.BlockSpec((1, f), lambda i: (0, 0)),
            pl.BlockSpec((1, f), lambda i: (0, 0)),
        ],
        out_specs=[
            pl.BlockSpec((1, f), lambda i: (0, 0)),
            pl.BlockSpec((1, f), lambda i: (0, 0)),
        ],
        out_shape=[
            jax.ShapeDtypeStruct((1, f), f32),
            jax.ShapeDtypeStruct((1, f), f32),
        ],
        scratch_shapes=[pltpu.VMEM((2, f), f32)],
    )(x, gamma2, beta2)

    src3 = edge_index[0].reshape(nsteps, 1, bt * e_per)
    dst3 = edge_index[1].reshape(nsteps, 1, bt * e_per)
    ew2 = edge_weight.reshape(1, e_per)

    c2 = pl.pallas_call(
        functools.partial(_gcn_kernel, bt=bt, n_per=n_per, e_per=e_per),
        grid=(nsteps,),
        in_specs=[
            pl.BlockSpec((rows, f), lambda i: (i, 0)),
            pl.BlockSpec((1, 1, bt * e_per), lambda i: (i, 0, 0)),
            pl.BlockSpec((1, 1, bt * e_per), lambda i: (i, 0, 0)),
            pl.BlockSpec((1, e_per), lambda i: (0, 0)),
            pl.BlockSpec((1, f), lambda i: (0, 0)),
            pl.BlockSpec((1, f), lambda i: (0, 0)),
            pl.BlockSpec((f, g1), lambda i: (0, 0)),
            pl.BlockSpec((1, g1), lambda i: (0, 0)),
            pl.BlockSpec((g1, g2), lambda i: (0, 0)),
            pl.BlockSpec((1, g2), lambda i: (0, 0)),
        ],
        out_specs=pl.BlockSpec((rows, g2), lambda i: (i, 0)),
        out_shape=jax.ShapeDtypeStruct((n, g2), f32),
    )(x, src3, dst3, ew2, scale, shift, W1, b1.reshape(1, g1), W2, b2.reshape(1, g2))

    g = c2.reshape(b_graphs, n_per, g2)

    fc1_n = fc1_W.shape[1]
    out_n = fc2_W.shape[1]
    fc_bt = 256 if b_graphs % 256 == 0 else b_graphs
    logits = pl.pallas_call(
        functools.partial(_fc_kernel, n_per=n_per),
        grid=(b_graphs // fc_bt,),
        in_specs=[
            pl.BlockSpec((fc_bt, n_per, g2), lambda i: (i, 0, 0)),
            pl.BlockSpec((n_per * g2, fc1_n), lambda i: (0, 0)),
            pl.BlockSpec((1, fc1_n), lambda i: (0, 0)),
            pl.BlockSpec((fc1_n, out_n), lambda i: (0, 0)),
            pl.BlockSpec((1, out_n), lambda i: (0, 0)),
        ],
        out_specs=pl.BlockSpec((fc_bt, out_n), lambda i: (i, 0)),
        out_shape=jax.ShapeDtypeStruct((b_graphs, out_n), f32),
    )(g, fc1_W, fc1_b.reshape(1, fc1_n), fc2_W, fc2_b.reshape(1, out_n))
    return logits


# speedup vs baseline: 1.8028x; 1.0252x over previous
import functools
import jax
import jax.numpy as jnp
from jax.experimental import pallas as pl
from jax.experimental.pallas import tpu as pltpu


def _celu(v):
    return jnp.where(v > 0, v, jnp.exp(jnp.minimum(v, 0.0)) - 1.0)


def _stats_kernel(x_ref, gamma_ref, beta_ref, scale_ref, shift_ref, acc_ref, *, nsteps, n_rows):
    i = pl.program_id(0)

    @pl.when(i == 0)
    def _init():
        acc_ref[...] = jnp.zeros_like(acc_ref)

    xb = x_ref[...]
    acc_ref[0:1, :] += jnp.sum(xb, axis=0, keepdims=True)
    acc_ref[1:2, :] += jnp.sum(xb * xb, axis=0, keepdims=True)

    @pl.when(i == nsteps - 1)
    def _fin():
        inv_n = 1.0 / n_rows
        mean = acc_ref[0:1, :] * inv_n
        var = acc_ref[1:2, :] * inv_n - mean * mean
        rstd = jax.lax.rsqrt(var + 1e-5)
        sc = gamma_ref[...] * rstd
        scale_ref[...] = sc
        shift_ref[...] = beta_ref[...] - mean * sc


def _gcn_kernel(x_ref, src_ref, dst_ref, ew_ref, scale_ref, shift_ref,
                w1_ref, b1_ref, w2_ref, b2_ref, out_ref, *, bt, n_per, e_per):
    f32 = jnp.float32
    xb = x_ref[...] * scale_ref[...] + shift_ref[...]
    h1 = jnp.dot(xb, w1_ref[...], preferred_element_type=f32)
    ew = ew_ref[...]
    iota_n = jax.lax.broadcasted_iota(jnp.int32, (n_per, e_per), 0)
    wmats = []
    dinvs = []
    for b in range(bt):
        src = src_ref[0, :, b * e_per:(b + 1) * e_per] & (n_per - 1)
        dst = dst_ref[0, :, b * e_per:(b + 1) * e_per] & (n_per - 1)
        dt = jnp.where(iota_n == dst, 1.0, 0.0)
        stw = jnp.where(iota_n == src, ew, 0.0)
        wmat = jax.lax.dot_general(dt, stw, (((1,), (1,)), ((), ())),
                                   preferred_element_type=f32)
        deg = jnp.sum(wmat, axis=1, keepdims=True) + 1.0
        dinvs.append(jnp.where(deg > 0, jax.lax.rsqrt(deg), 0.0))
        wmats.append(wmat)
    dinv = jnp.concatenate(dinvs, axis=0)
    t1 = dinv * h1
    u1 = jnp.concatenate(
        [jnp.dot(wmats[b], t1[b * n_per:(b + 1) * n_per, :], preferred_element_type=f32)
         for b in range(bt)], axis=0)
    c1 = _celu(dinv * (u1 + t1) + b1_ref[...])
    h2 = jnp.dot(c1, w2_ref[...], preferred_element_type=f32)
    t2 = dinv * h2
    u2 = jnp.concatenate(
        [jnp.dot(wmats[b], t2[b * n_per:(b + 1) * n_per, :], preferred_element_type=f32)
         for b in range(bt)], axis=0)
    out_ref[...] = _celu(dinv * (u2 + t2) + b2_ref[...])


def _fc_kernel(g_ref, w1_ref, b1_ref, w2_ref, b2_ref, out_ref, *, n_per):
    f32 = jnp.float32
    gv = jnp.concatenate([g_ref[:, nn, :] for nn in range(n_per)], axis=1)
    h = _celu(jnp.dot(gv, w1_ref[...], preferred_element_type=f32) + b1_ref[...])
    out_ref[...] = jnp.dot(h, w2_ref[...], preferred_element_type=f32) + b2_ref[...]


@jax.jit
def kernel(x, edge_index, edge_weight, bn_gamma, bn_beta, W1, b1, W2, b2, fc1_W, fc1_b, fc2_W, fc2_b):
    f32 = jnp.float32
    n, f = x.shape
    g1 = W1.shape[1]
    g2 = W2.shape[1]
    n_per = fc1_W.shape[0] // g2
    b_graphs = n // n_per
    e_per = edge_weight.shape[0]

    bt = 32 if b_graphs % 32 == 0 else 1
    nsteps = b_graphs // bt
    rows = bt * n_per

    gamma2 = bn_gamma.reshape(1, f)
    beta2 = bn_beta.reshape(1, f)
    srows = 16384 if n % 16384 == 0 else rows
    s_steps = n // srows
    scale, shift = pl.pallas_call(
        functools.partial(_stats_kernel, nsteps=s_steps, n_rows=float(n)),
        grid=(s_steps,),
        in_specs=[
            pl.BlockSpec((srows, f), lambda i: (i, 0)),
            pl.BlockSpec((1, f), lambda i: (0, 0)),
            pl.BlockSpec((1, f), lambda i: (0, 0)),
        ],
        out_specs=[
            pl.BlockSpec((1, f), lambda i: (0, 0)),
            pl.BlockSpec((1, f), lambda i: (0, 0)),
        ],
        out_shape=[
            jax.ShapeDtypeStruct((1, f), f32),
            jax.ShapeDtypeStruct((1, f), f32),
        ],
        scratch_shapes=[pltpu.VMEM((2, f), f32)],
    )(x, gamma2, beta2)

    src3 = edge_index[0].reshape(nsteps, 1, bt * e_per)
    dst3 = edge_index[1].reshape(nsteps, 1, bt * e_per)
    ew2 = edge_weight.reshape(1, e_per)

    c2 = pl.pallas_call(
        functools.partial(_gcn_kernel, bt=bt, n_per=n_per, e_per=e_per),
        grid=(nsteps,),
        in_specs=[
            pl.BlockSpec((rows, f), lambda i: (i, 0)),
            pl.BlockSpec((1, 1, bt * e_per), lambda i: (i, 0, 0)),
            pl.BlockSpec((1, 1, bt * e_per), lambda i: (i, 0, 0)),
            pl.BlockSpec((1, e_per), lambda i: (0, 0)),
            pl.BlockSpec((1, f), lambda i: (0, 0)),
            pl.BlockSpec((1, f), lambda i: (0, 0)),
            pl.BlockSpec((f, g1), lambda i: (0, 0)),
            pl.BlockSpec((1, g1), lambda i: (0, 0)),
            pl.BlockSpec((g1, g2), lambda i: (0, 0)),
            pl.BlockSpec((1, g2), lambda i: (0, 0)),
        ],
        out_specs=pl.BlockSpec((rows, g2), lambda i: (i, 0)),
        out_shape=jax.ShapeDtypeStruct((n, g2), f32),
    )(x, src3, dst3, ew2, scale, shift, W1, b1.reshape(1, g1), W2, b2.reshape(1, g2))

    g = c2.reshape(b_graphs, n_per, g2)

    fc1_n = fc1_W.shape[1]
    out_n = fc2_W.shape[1]
    fc_bt = 512 if b_graphs % 512 == 0 else b_graphs
    logits = pl.pallas_call(
        functools.partial(_fc_kernel, n_per=n_per),
        grid=(b_graphs // fc_bt,),
        in_specs=[
            pl.BlockSpec((fc_bt, n_per, g2), lambda i: (i, 0, 0)),
            pl.BlockSpec((n_per * g2, fc1_n), lambda i: (0, 0)),
            pl.BlockSpec((1, fc1_n), lambda i: (0, 0)),
            pl.BlockSpec((fc1_n, out_n), lambda i: (0, 0)),
            pl.BlockSpec((1, out_n), lambda i: (0, 0)),
        ],
        out_specs=pl.BlockSpec((fc_bt, out_n), lambda i: (i, 0)),
        out_shape=jax.ShapeDtypeStruct((b_graphs, out_n), f32),
    )(g, fc1_W, fc1_b.reshape(1, fc1_n), fc2_W, fc2_b.reshape(1, out_n))
    return logits
